# Initial kernel scaffold; baseline (speedup 1.0000x reference)
import jax
import jax.numpy as jnp
from jax.experimental import pallas as pl


def kernel(text, emb_table, W1, b1, W2, b2):
    raise NotImplementedError("write your pallas kernel here")


# speedup vs baseline: 2.5145x; 2.5145x over previous
import functools

import jax
import jax.numpy as jnp
from jax import lax
from jax.experimental import pallas as pl
from jax.experimental.pallas import tpu as pltpu
from jax.experimental.pallas import tpu_sc as plsc

B = 16384
HL = 50
D = 32
H = 64
O = 2

NC = 2
NS = 16
NW = NC * NS

ENTRIES = B * HL
IDXW = 128
IDX_ROWS = ENTRIES // IDXW
ROWS_PER_W = IDX_ROWS // NW
K = 8
CHUNKS = ROWS_PER_W // K
B_PER_W = B // NW
B_PER_SC = B // NC


def _sc_pool(emb_table, src_idx, dst_idx, zeros):
    mesh = plsc.VectorSubcoreMesh(core_axis_name="c", subcore_axis_name="s")

    @functools.partial(
        pl.kernel,
        out_type=jax.ShapeDtypeStruct((B, D), jnp.float32),
        mesh=mesh,
        compiler_params=pltpu.CompilerParams(use_tc_tiling_on_sc=False),
        scratch_types=[
            pltpu.VMEM((K, IDXW), jnp.int32),
            pltpu.VMEM((K, IDXW), jnp.int32),
            pltpu.VMEM((K, IDXW, D), jnp.float32),
            pltpu.VMEM((B_PER_W, D), jnp.float32),
            pltpu.VMEM_SHARED((B_PER_SC, D), jnp.float32),
            pltpu.SemaphoreType.DMA,
        ],
    )
    def k(table_hbm, src_hbm, dst_hbm, zero_hbm, out_hbm,
          idx_s, idx_d, rows, stage, acc, sem):
        c = lax.axis_index("c")
        s = lax.axis_index("s")
        wid = c * NS + s

        pltpu.sync_copy(zero_hbm, acc.at[pl.ds(s * B_PER_W, B_PER_W)])
        plsc.subcore_barrier()

        def chunk(i, _):
            row0 = wid * ROWS_PER_W + i * K
            pltpu.sync_copy(src_hbm.at[pl.ds(row0, K)], idx_s)
            pltpu.sync_copy(dst_hbm.at[pl.ds(row0, K)], idx_d)
            descs = []
            for j in range(K):
                descs.append(
                    pltpu.async_copy(table_hbm.at[idx_s.at[j]], rows.at[j], sem))
            for d_ in descs:
                d_.wait()
            for j in range(K):
                pltpu.sync_copy(rows.at[j], acc.at[idx_d.at[j]], add=True)
            return 0

        lax.fori_loop(0, CHUNKS, chunk, 0)

        plsc.subcore_barrier()
        pltpu.sync_copy(acc.at[pl.ds(s * B_PER_W, B_PER_W)], stage)
        pltpu.sync_copy(stage, out_hbm.at[pl.ds(wid * B_PER_W, B_PER_W)])

    return k(emb_table, src_idx, dst_idx, zeros)


def _mlp_body(p_ref, w1_ref, b1_ref, w2_ref, b2_ref, o_ref):
    p = p_ref[...] * (1.0 / HL)
    h = jnp.dot(p, w1_ref[...], preferred_element_type=jnp.float32)
    h = jnp.maximum(h + b1_ref[...], 0.0)
    o = jnp.dot(h, w2_ref[...], preferred_element_type=jnp.float32)
    o_ref[...] = o + b2_ref[...]


def _tc_mlp(pooled, W1, b1, W2, b2):
    GB = 2048
    grid = (B // GB,)
    return pl.pallas_call(
        _mlp_body,
        grid=grid,
        in_specs=[
            pl.BlockSpec((GB, D), lambda i: (i, 0)),
            pl.BlockSpec((D, H), lambda i: (0, 0)),
            pl.BlockSpec((1, H), lambda i: (0, 0)),
            pl.BlockSpec((H, O), lambda i: (0, 0)),
            pl.BlockSpec((1, O), lambda i: (0, 0)),
        ],
        out_specs=pl.BlockSpec((GB, O), lambda i: (i, 0)),
        out_shape=jax.ShapeDtypeStruct((B, O), jnp.float32),
    )(pooled, W1, b1, W2, b2)


def kernel(text, emb_table, W1, b1, W2, b2):
    src_idx = text.astype(jnp.int32).reshape(IDX_ROWS, IDXW)
    dst_idx = jnp.repeat(
        jnp.arange(B, dtype=jnp.int32) % B_PER_SC, HL).reshape(IDX_ROWS, IDXW)
    zeros = jnp.zeros((B_PER_W, D), jnp.float32)
    pooled = _sc_pool(emb_table, src_idx, dst_idx, zeros)
    return _tc_mlp(pooled, W1.astype(jnp.float32), b1.reshape(1, H),
                   W2.astype(jnp.float32), b2.reshape(1, O))


# speedup vs baseline: 2.6730x; 1.0630x over previous
import functools

import jax
import jax.numpy as jnp
from jax import lax
from jax.experimental import pallas as pl
from jax.experimental.pallas import tpu as pltpu
from jax.experimental.pallas import tpu_sc as plsc

B = 16384
HL = 50
D = 32
H = 64
O = 2

NC = 2
NS = 16
NW = NC * NS

IDXW = 128
ROWS_PER_BLK = B // IDXW
RPW = ROWS_PER_BLK // NW
EPC = RPW * IDXW
B_PER_W = B // NW
B_PER_SC = B // NC
PAIRS = HL // 2


def _sc_pool(emb_table, src_idx, dst_base, zeros):
    mesh = plsc.VectorSubcoreMesh(core_axis_name="c", subcore_axis_name="s")

    @functools.partial(
        pl.kernel,
        out_type=jax.ShapeDtypeStruct((B, D), jnp.float32),
        mesh=mesh,
        compiler_params=pltpu.CompilerParams(use_tc_tiling_on_sc=False),
        scratch_types=[
            pltpu.VMEM((RPW, IDXW), jnp.int32),
            pltpu.VMEM((RPW, IDXW), jnp.int32),
            pltpu.VMEM((RPW, IDXW, D), jnp.float32),
            pltpu.VMEM((RPW, IDXW, D), jnp.float32),
            pltpu.VMEM((RPW, IDXW), jnp.int32),
            pltpu.VMEM((B_PER_W, D), jnp.float32),
            pltpu.VMEM_SHARED((B_PER_SC, D), jnp.float32),
            pltpu.SemaphoreType.DMA,
            pltpu.SemaphoreType.DMA,
            pltpu.SemaphoreType.DMA,
            pltpu.SemaphoreType.DMA,
        ],
    )
    def k(table_hbm, src_hbm, dstb_hbm, zero_hbm, out_hbm,
          idx_a, idx_b, rows_a, rows_b, dst_v, stage, acc,
          sem_ga, sem_gb, sem_sa, sem_sb):
        c = lax.axis_index("c")
        s = lax.axis_index("s")
        wid = c * NS + s

        pltpu.sync_copy(dstb_hbm.at[s], dst_v)
        pltpu.sync_copy(zero_hbm, acc.at[pl.ds(s * B_PER_W, B_PER_W)])
        plsc.subcore_barrier()

        def fire_gathers(blk, idx, rows, sem):
            pltpu.sync_copy(src_hbm.at[pl.ds(blk * ROWS_PER_BLK + wid * RPW,
                                             RPW)], idx)
            for j in range(RPW):
                pltpu.async_copy(table_hbm.at[idx.at[j]], rows.at[j], sem)

        def drain_gathers(idx, rows, sem):
            for j in range(RPW):
                pltpu.make_async_copy(table_hbm.at[idx.at[j]], rows.at[j],
                                      sem).wait()

        def fire_scatters(rows, sem):
            for j in range(RPW):
                pltpu.async_copy(rows.at[j], acc.at[dst_v.at[j]], sem,
                                 add=True)

        def drain_scatters(rows, sem):
            for j in range(RPW):
                pltpu.make_async_copy(rows.at[j], acc.at[dst_v.at[j]],
                                      sem).wait()

        fire_gathers(0, idx_a, rows_a, sem_ga)

        def pair(i, _):
            b_blk = 2 * i + 1
            drain_gathers(idx_a, rows_a, sem_ga)
            fire_scatters(rows_a, sem_sa)

            @pl.when(i > 0)
            def _():
                drain_scatters(rows_b, sem_sb)

            fire_gathers(b_blk, idx_b, rows_b, sem_gb)
            drain_gathers(idx_b, rows_b, sem_gb)
            fire_scatters(rows_b, sem_sb)
            drain_scatters(rows_a, sem_sa)

            @pl.when(i < PAIRS - 1)
            def _():
                fire_gathers(b_blk + 1, idx_a, rows_a, sem_ga)

            return 0

        lax.fori_loop(0, PAIRS, pair, 0)
        drain_scatters(rows_b, sem_sb)

        plsc.subcore_barrier()
        pltpu.sync_copy(acc.at[pl.ds(s * B_PER_W, B_PER_W)], stage)
        pltpu.sync_copy(stage, out_hbm.at[pl.ds(wid * B_PER_W, B_PER_W)])

    return k(emb_table, src_idx, dst_base, zeros)


def _mlp_body(p_ref, w1_ref, b1_ref, w2_ref, b2_ref, o_ref):
    p = p_ref[...] * (1.0 / HL)
    h = jnp.dot(p, w1_ref[...], preferred_element_type=jnp.float32)
    h = jnp.maximum(h + b1_ref[...], 0.0)
    o = jnp.dot(h, w2_ref[...], preferred_element_type=jnp.float32)
    o_ref[...] = o + b2_ref[...]


def _tc_mlp(pooled, W1, b1, W2, b2):
    GB = 2048
    grid = (B // GB,)
    return pl.pallas_call(
        _mlp_body,
        grid=grid,
        in_specs=[
            pl.BlockSpec((GB, D), lambda i: (i, 0)),
            pl.BlockSpec((D, H), lambda i: (0, 0)),
            pl.BlockSpec((1, H), lambda i: (0, 0)),
            pl.BlockSpec((H, O), lambda i: (0, 0)),
            pl.BlockSpec((1, O), lambda i: (0, 0)),
        ],
        out_specs=pl.BlockSpec((GB, O), lambda i: (i, 0)),
        out_shape=jax.ShapeDtypeStruct((B, O), jnp.float32),
    )(pooled, W1, b1, W2, b2)


def kernel(text, emb_table, W1, b1, W2, b2):
    src_idx = text.T.astype(jnp.int32).reshape(HL * ROWS_PER_BLK, IDXW)
    dst_base = jnp.arange(B_PER_SC, dtype=jnp.int32).reshape(NS, RPW, IDXW)
    zeros = jnp.zeros((B_PER_W, D), jnp.float32)
    pooled = _sc_pool(emb_table, src_idx, dst_base, zeros)
    return _tc_mlp(pooled, W1.astype(jnp.float32), b1.reshape(1, H),
                   W2.astype(jnp.float32), b2.reshape(1, O))
